# baseline (speedup 1.0000x reference)
import jax
import jax.numpy as jnp
from jax.experimental import pallas as pl

_G = 4


def _body(xg_ref, A_ref, W_in_ref, b_in_ref, W_ref, b_ref, W_out_ref,
          b_out_ref, out_ref):
    GC, Tg, F = xg_ref.shape
    H = W_in_ref.shape[1]
    L = A_ref.shape[0]
    C = A_ref.shape[1]
    G = GC // C

    x = xg_ref[...].reshape(GC * Tg, F).astype(jnp.bfloat16)
    h = jnp.maximum(
        jnp.dot(x, W_in_ref[...].astype(jnp.bfloat16),
                preferred_element_type=jnp.float32)
        + b_in_ref[...], 0.0)

    row_g = jax.lax.broadcasted_iota(jnp.int32, (GC, GC), 0) // C
    col_g = jax.lax.broadcasted_iota(jnp.int32, (GC, GC), 1) // C
    diag = row_g == col_g

    for l in range(L):
        a = A_ref[l]
        a = a - jnp.max(a, axis=-1, keepdims=True)
        e = jnp.exp(a)
        An = e / jnp.sum(e, axis=-1, keepdims=True)
        An_bd = jnp.where(diag, jnp.tile(An, (G, G)), 0.0)
        m = jnp.dot(An_bd.astype(jnp.bfloat16),
                    h.astype(jnp.bfloat16).reshape(GC, Tg * H),
                    preferred_element_type=jnp.float32)
        h = jnp.maximum(
            jnp.dot(m.reshape(GC * Tg, H).astype(jnp.bfloat16),
                    W_ref[l].astype(jnp.bfloat16),
                    preferred_element_type=jnp.float32) + b_ref[l], 0.0)

    feat = jnp.mean(h.reshape(G, C, Tg, H), axis=1)
    out_ref[...] = (
        jnp.dot(feat.reshape(G * Tg, H).astype(jnp.bfloat16),
                W_out_ref[...].astype(jnp.bfloat16),
                preferred_element_type=jnp.float32) + b_out_ref[...])


def kernel(x, W_in, b_in, A, W, b, W_out, b_out):
    B, C, F = x.shape
    H = W_in.shape[1]
    K = W_out.shape[1]

    T = 1024
    G = _G
    Tg = T // G
    assert B % T == 0 and T % G == 0
    n_tiles = B // T

    xg = jnp.transpose(x.reshape(n_tiles, G, Tg, C, F), (0, 1, 3, 2, 4))
    xg = xg.reshape(n_tiles * G * C, Tg, F)

    return pl.pallas_call(
        _body,
        grid=(n_tiles,),
        in_specs=[
            pl.BlockSpec((G * C, Tg, F), lambda i: (i, 0, 0)),
            pl.BlockSpec(A.shape, lambda i: (0, 0, 0)),
            pl.BlockSpec(W_in.shape, lambda i: (0, 0)),
            pl.BlockSpec((1, H), lambda i: (0, 0)),
            pl.BlockSpec(W.shape, lambda i: (0, 0, 0)),
            pl.BlockSpec(b.shape, lambda i: (0, 0)),
            pl.BlockSpec(W_out.shape, lambda i: (0, 0)),
            pl.BlockSpec((1, K), lambda i: (0, 0)),
        ],
        out_specs=pl.BlockSpec((T, K), lambda i: (i, 0)),
        out_shape=jax.ShapeDtypeStruct((B, K), jnp.float32),
    )(xg, A, W_in, b_in.reshape(1, H), W, b, W_out, b_out.reshape(1, K))


# speedup vs baseline: 1.0011x; 1.0011x over previous
import jax
import jax.numpy as jnp
from jax.experimental import pallas as pl
from jax.experimental.pallas import tpu as pltpu

_G = 4


def _body(xg_ref, A_ref, W_in_ref, b_in_ref, W_ref, b_ref, W_out_ref,
          b_out_ref, out_ref):
    GC, Tg, F = xg_ref.shape
    H = W_in_ref.shape[1]
    L = A_ref.shape[0]
    C = A_ref.shape[1]
    G = GC // C

    x = xg_ref[...].reshape(GC * Tg, F).astype(jnp.bfloat16)
    h = jnp.maximum(
        jnp.dot(x, W_in_ref[...].astype(jnp.bfloat16),
                preferred_element_type=jnp.float32)
        + b_in_ref[...], 0.0)

    row_g = jax.lax.broadcasted_iota(jnp.int32, (GC, GC), 0) // C
    col_g = jax.lax.broadcasted_iota(jnp.int32, (GC, GC), 1) // C
    diag = row_g == col_g

    for l in range(L):
        a = A_ref[l]
        a = a - jnp.max(a, axis=-1, keepdims=True)
        e = jnp.exp(a)
        An = e / jnp.sum(e, axis=-1, keepdims=True)
        An_bd = jnp.where(diag, jnp.tile(An, (G, G)), 0.0)
        m = jnp.dot(An_bd.astype(jnp.bfloat16),
                    h.astype(jnp.bfloat16).reshape(GC, Tg * H),
                    preferred_element_type=jnp.float32)
        h = jnp.maximum(
            jnp.dot(m.reshape(GC * Tg, H).astype(jnp.bfloat16),
                    W_ref[l].astype(jnp.bfloat16),
                    preferred_element_type=jnp.float32) + b_ref[l], 0.0)

    feat = jnp.mean(h.reshape(G, C, Tg, H), axis=1)
    out_ref[...] = (
        jnp.dot(feat.reshape(G * Tg, H).astype(jnp.bfloat16),
                W_out_ref[...].astype(jnp.bfloat16),
                preferred_element_type=jnp.float32) + b_out_ref[...])


def kernel(x, W_in, b_in, A, W, b, W_out, b_out):
    B, C, F = x.shape
    H = W_in.shape[1]
    K = W_out.shape[1]

    T = 1024
    G = _G
    Tg = T // G
    assert B % T == 0 and T % G == 0
    n_tiles = B // T

    xg = jnp.transpose(x.reshape(n_tiles, G, Tg, C, F), (0, 1, 3, 2, 4))
    xg = xg.reshape(n_tiles * G * C, Tg, F)

    return pl.pallas_call(
        _body,
        grid=(n_tiles,),
        in_specs=[
            pl.BlockSpec((G * C, Tg, F), lambda i: (i, 0, 0)),
            pl.BlockSpec(A.shape, lambda i: (0, 0, 0)),
            pl.BlockSpec(W_in.shape, lambda i: (0, 0)),
            pl.BlockSpec((1, H), lambda i: (0, 0)),
            pl.BlockSpec(W.shape, lambda i: (0, 0, 0)),
            pl.BlockSpec(b.shape, lambda i: (0, 0)),
            pl.BlockSpec(W_out.shape, lambda i: (0, 0)),
            pl.BlockSpec((1, K), lambda i: (0, 0)),
        ],
        out_specs=pl.BlockSpec((T, K), lambda i: (i, 0)),
        out_shape=jax.ShapeDtypeStruct((B, K), jnp.float32),
        compiler_params=pltpu.CompilerParams(
            dimension_semantics=("parallel",)),
    )(xg, A, W_in, b_in.reshape(1, H), W, b, W_out, b_out.reshape(1, K))
